# baseline (speedup 1.0000x reference)
import functools

import jax
import jax.numpy as jnp
from jax.experimental import pallas as pl
from jax.experimental.pallas import tpu as pltpu

_NUM_CLASSES = 80
_LAMBDA_COORD = 5.0
_BLOCK_N = 3840


def _softplus(x):
    return jnp.maximum(x, 0.0) + jnp.log1p(jnp.exp(-jnp.abs(x)))


def _arctan(x):
    sgn = jnp.where(x < 0.0, -1.0, 1.0)
    ax = jnp.abs(x)
    inv = ax > 1.0
    z = jnp.where(inv, 1.0 / ax, ax)
    z2 = z * z
    p = -0.0161657367 + z2 * 0.0028662257
    p = 0.0429096138 + z2 * p
    p = -0.0752896400 + z2 * p
    p = 0.1065626393 + z2 * p
    p = -0.1420889944 + z2 * p
    p = 0.1999355085 + z2 * p
    p = -0.3333314528 + z2 * p
    r = z * (1.0 + z2 * p)
    r = jnp.where(inv, jnp.pi / 2 - r, r)
    return sgn * r


def _t(x):
    return jnp.swapaxes(x, -1, -2)


def _phase1_body(nblocks, n_total, pb_ref, pc_ref, tb_ref,
                 gidx_ref, tidx_ref, valid_ref, obj_ref,
                 rv, ri, rpc, ssum):
    T = tb_ref.shape[1]
    Nb = pb_ref.shape[2]
    nb = pl.program_id(1)

    @pl.when(nb == 0)
    def _():
        rv[...] = jnp.full_like(rv[...], -1.0)
        ri[...] = jnp.zeros_like(ri[...])
        rpc[...] = jnp.zeros_like(rpc[...])
        ssum[...] = jnp.zeros_like(ssum[...])

    px1 = pb_ref[0, 0:1, :]
    py1 = pb_ref[0, 1:2, :]
    px2 = pb_ref[0, 2:3, :]
    py2 = pb_ref[0, 3:4, :]
    tb = tb_ref[0]
    tx1, ty1 = tb[:, 0:1], tb[:, 1:2]
    tx2, ty2 = tb[:, 2:3], tb[:, 3:4]

    pa = (px2 - px1) * (py2 - py1)
    ta = (tx2 - tx1) * (ty2 - ty1)
    ix1 = jnp.maximum(px1, tx1)
    iy1 = jnp.maximum(py1, ty1)
    ix2 = jnp.minimum(px2, tx2)
    iy2 = jnp.minimum(py2, ty2)
    inter = jnp.maximum(ix2 - ix1, 0.0) * jnp.maximum(iy2 - iy1, 0.0)
    union = pa + ta - inter
    iou = inter / (union + 1e-6)

    m = jnp.max(iou, axis=1, keepdims=True)
    lane = (jax.lax.broadcasted_iota(jnp.int32, (T, Nb), 1)
            + nb * Nb)
    li = jnp.min(jnp.where(iou == m, lane, jnp.int32(2 ** 30)),
                 axis=1, keepdims=True)
    pc = pc_ref[0]
    lpc = jnp.max(jnp.where(lane == li, pc, -jnp.inf),
                  axis=1, keepdims=True)

    better = (m > rv[...]) | ((m == rv[...]) & (li < ri[...]))
    rv[...] = jnp.where(better, m, rv[...])
    ri[...] = jnp.where(better, li, ri[...])
    rpc[...] = jnp.where(better, lpc, rpc[...])
    ssum[...] = ssum[...] + jnp.sum(_softplus(pc), axis=1, keepdims=True)

    @pl.when(nb == nblocks - 1)
    def _():
        ii = jax.lax.broadcasted_iota(jnp.int32, (T, T), 0)
        jj = jax.lax.broadcasted_iota(jnp.int32, (T, T), 1)
        v_col = ri[...]
        v_row = _t(v_col)
        dup = (v_col == v_row) & (jj < ii)
        first = ~jnp.any(dup, axis=1, keepdims=True)
        c_col = jnp.where(first, v_col, n_total)
        c_row = _t(c_col)
        less = (c_row < c_col)
        eqlt = (c_row == c_col) & (jj < ii)
        r_col = jnp.sum(less.astype(jnp.int32) + eqlt.astype(jnp.int32),
                        axis=1, keepdims=True)
        r_row = _t(r_col)
        pos_col = jnp.sum(jnp.where(r_row == ii, c_row, 0),
                          axis=1, keepdims=True)
        valid_col = (pos_col < n_total)
        g_col = jnp.minimum(pos_col, n_total - 1)
        t_col = jnp.minimum(pos_col, T - 1)
        pc_first = jnp.sum(jnp.where(first, rpc[...], 0.0),
                           axis=0, keepdims=True)
        obj = (ssum[...] - pc_first) * (1.0 / n_total)
        gidx_ref[0] = _t(g_col)
        tidx_ref[0] = _t(t_col)
        valid_ref[0] = _t(valid_col.astype(jnp.float32))
        obj_ref[0] = jnp.broadcast_to(obj, (1, T))


def _phase2_body(nimages, gidx_sm, tidx_sm,
                 pb_ref, cls_hbm, tb_ref, lab_ref, tidxv_ref, valid_ref,
                 obj_ref, out_ref, pbg, tbg, clsg, sem):
    T = tb_ref.shape[1]
    C = clsg.shape[1]
    b = pl.program_id(0)

    for t in range(T):
        g = gidx_sm[b, t]
        pltpu.make_async_copy(cls_hbm.at[b, pl.ds(g, 1), :],
                              clsg.at[pl.ds(t, 1), :], sem).start()
    for t in range(T):
        g = gidx_sm[b, t]
        pbg[pl.ds(t, 1), :] = pb_ref[0, pl.ds(g, 1), :]
        ti = tidx_sm[b, t]
        tbg[pl.ds(t, 1), :] = tb_ref[0, pl.ds(ti, 1), :]
    for t in range(T):
        g = gidx_sm[b, t]
        pltpu.make_async_copy(cls_hbm.at[b, pl.ds(g, 1), :],
                              clsg.at[pl.ds(t, 1), :], sem).wait()

    p = pbg[...]
    q = tbg[...]
    px1c, py1c = p[:, 0:1], p[:, 1:2]
    px2c, py2c = p[:, 2:3], p[:, 3:4]
    px1r, py1r = _t(px1c), _t(py1c)
    px2r, py2r = _t(px2c), _t(py2c)
    qx1r, qy1r = _t(q[:, 0:1]), _t(q[:, 1:2])
    qx2r, qy2r = _t(q[:, 2:3]), _t(q[:, 3:4])

    a1 = (px2c - px1c) * (py2c - py1c)
    a2 = (qx2r - qx1r) * (qy2r - qy1r)
    ix1 = jnp.maximum(px1c, qx1r)
    iy1 = jnp.maximum(py1c, qy1r)
    ix2 = jnp.minimum(px2c, qx2r)
    iy2 = jnp.minimum(py2c, qy2r)
    inter = jnp.maximum(ix2 - ix1, 0.0) * jnp.maximum(iy2 - iy1, 0.0)
    iou = inter / (a1 + a2 - inter + 1e-6)

    c_diag = ((jnp.maximum(px2r, qx2r) - jnp.minimum(px1r, qx1r)) ** 2
              + (jnp.maximum(py2r, qy2r) - jnp.minimum(py1r, qy1r)) ** 2)
    center = (((px1r + px2r) / 2 - (qx1r + qx2r) / 2) ** 2
              + ((py1r + py2r) / 2 - (qy1r + qy2r) / 2) ** 2)
    w1, h1 = px2r - px1r, py2r - py1r
    w2, h2 = qx2r - qx1r, qy2r - qy1r
    v = (4.0 / jnp.pi ** 2) * (_arctan(w2 / h2) - _arctan(w1 / h1)) ** 2
    alpha = v / (1.0 - iou + v + 1e-6)
    closs = 1.0 - (iou - center / c_diag - alpha * v)

    vrow = valid_ref[0]
    vcol = _t(vrow)
    m2 = vcol * vrow
    m2sum = jnp.sum(jnp.sum(m2, axis=1, keepdims=True), axis=0, keepdims=True)
    bsum = jnp.sum(jnp.sum(closs * m2, axis=1, keepdims=True),
                   axis=0, keepdims=True)
    box_l = bsum / jnp.maximum(m2sum, 1.0)

    x = clsg[...]
    rs = jnp.sum(_softplus(x), axis=1, keepdims=True)
    jjT = jax.lax.broadcasted_iota(jnp.int32, (T, T), 1)
    tic = _t(tidxv_ref[0])
    labg = jnp.sum(jnp.where(tic == jjT, lab_ref[0], 0),
                   axis=1, keepdims=True)
    cc = jax.lax.broadcasted_iota(jnp.int32, (T, C), 1)
    sel = jnp.sum(jnp.where(cc == labg, x, 0.0), axis=1, keepdims=True)
    rowloss = rs - sel
    cls_sum = jnp.sum(rowloss * vcol, axis=0, keepdims=True)
    nvalid = jnp.sum(vrow, axis=1, keepdims=True)
    cls_l = cls_sum / jnp.maximum(nvalid * C, 1.0)

    obj_l = obj_ref[0][0:1, 0:1]

    @pl.when(b == 0)
    def _():
        out_ref[...] = jnp.zeros_like(out_ref[...])
    out_ref[...] = out_ref[...] + (
        (_LAMBDA_COORD * box_l + obj_l + cls_l) * (1.0 / nimages))


def kernel(pred_boxes, pred_conf, pred_cls, target_boxes, target_labels,
           anchors):
    del anchors
    B, N, _ = pred_boxes.shape
    T = target_boxes.shape[1]
    C = pred_cls.shape[-1]
    nblocks = N // _BLOCK_N

    pbT = jnp.transpose(pred_boxes, (0, 2, 1))
    pc3 = jnp.transpose(pred_conf, (0, 2, 1))
    lab3 = target_labels.reshape(B, 1, T).astype(jnp.int32)

    p1 = pl.pallas_call(
        functools.partial(_phase1_body, nblocks, N),
        grid=(B, nblocks),
        in_specs=[
            pl.BlockSpec((1, 4, _BLOCK_N), lambda b, nb: (b, 0, nb)),
            pl.BlockSpec((1, 1, _BLOCK_N), lambda b, nb: (b, 0, nb)),
            pl.BlockSpec((1, T, 4), lambda b, nb: (b, 0, 0)),
        ],
        out_specs=[
            pl.BlockSpec((1, 1, T), lambda b, nb: (b, 0, 0)),
            pl.BlockSpec((1, 1, T), lambda b, nb: (b, 0, 0)),
            pl.BlockSpec((1, 1, T), lambda b, nb: (b, 0, 0)),
            pl.BlockSpec((1, 1, T), lambda b, nb: (b, 0, 0)),
        ],
        out_shape=[
            jax.ShapeDtypeStruct((B, 1, T), jnp.int32),
            jax.ShapeDtypeStruct((B, 1, T), jnp.int32),
            jax.ShapeDtypeStruct((B, 1, T), jnp.float32),
            jax.ShapeDtypeStruct((B, 1, T), jnp.float32),
        ],
        scratch_shapes=[
            pltpu.VMEM((T, 1), jnp.float32),
            pltpu.VMEM((T, 1), jnp.int32),
            pltpu.VMEM((T, 1), jnp.float32),
            pltpu.VMEM((1, 1), jnp.float32),
        ],
        compiler_params=pltpu.CompilerParams(
            dimension_semantics=("parallel", "arbitrary")),
        name="yolo_phase1",
    )(pbT, pc3, target_boxes)
    gidx3, tidx3, valid3, obj3 = p1

    out = pl.pallas_call(
        functools.partial(_phase2_body, B),
        grid_spec=pltpu.PrefetchScalarGridSpec(
            num_scalar_prefetch=2,
            grid=(B,),
            in_specs=[
                pl.BlockSpec((1, N, 4), lambda b, *_: (b, 0, 0)),
                pl.BlockSpec(memory_space=pl.ANY),
                pl.BlockSpec((1, T, 4), lambda b, *_: (b, 0, 0)),
                pl.BlockSpec((1, 1, T), lambda b, *_: (b, 0, 0)),
                pl.BlockSpec((1, 1, T), lambda b, *_: (b, 0, 0)),
                pl.BlockSpec((1, 1, T), lambda b, *_: (b, 0, 0)),
                pl.BlockSpec((1, 1, T), lambda b, *_: (b, 0, 0)),
            ],
            out_specs=pl.BlockSpec((1, 1), lambda b, *_: (0, 0)),
            scratch_shapes=[
                pltpu.VMEM((T, 4), jnp.float32),
                pltpu.VMEM((T, 4), jnp.float32),
                pltpu.VMEM((T, C), jnp.float32),
                pltpu.SemaphoreType.DMA,
            ],
        ),
        out_shape=jax.ShapeDtypeStruct((1, 1), jnp.float32),
        compiler_params=pltpu.CompilerParams(
            dimension_semantics=("arbitrary",)),
        name="yolo_phase2",
    )(gidx3.reshape(B, T), tidx3.reshape(B, T),
      pred_boxes, pred_cls, target_boxes, lab3, tidx3, valid3, obj3)
    return out[0, 0]
